# Initial kernel scaffold; baseline (speedup 1.0000x reference)
import jax
import jax.numpy as jnp
from jax.experimental import pallas as pl


def kernel(x_note, x_user, edge_index_un, edge_index_nu, edge_attr_un, edge_attr_nu, W_l1_un, b_l1_un, W_r1_un, W_l1_nu, b_l1_nu, W_r1_nu, W_l2_un, b_l2_un, W_r2_un, W_l2_nu, b_l2_nu, W_r2_nu):
    raise NotImplementedError("write your pallas kernel here")


# speedup vs baseline: 4.0607x; 4.0607x over previous
import functools

import jax
import jax.numpy as jnp
from jax import lax
from jax.experimental import pallas as pl
from jax.experimental.pallas import tpu as pltpu
from jax.experimental.pallas import tpu_sc as plsc

N_NOTE = 50000
N_USER = 50000
E = 400000
D = 128

NC = 2
NS = 16
L = 16

NSWEEP = 2
R = 12544
NP = NSWEEP * NC * R
TRASH = 128
ACC_ROWS = R + TRASH
EPT = E // NS
B = 4096
NBLK = (EPT + B - 1) // B
STG = B + L
CAPB = B + 64
FLUSH = 64
ZROWS = ACC_ROWS // NS
OROWS = R // NS

_MESH = plsc.VectorSubcoreMesh(core_axis_name="c", subcore_axis_name="s")
_SC_PARAMS = pltpu.CompilerParams(needs_layout_passes=False)
_SCRATCH = [
    pltpu.VMEM((STG,), jnp.int32),
    pltpu.VMEM((STG,), jnp.int32),
    pltpu.VMEM((CAPB,), jnp.int32),
    pltpu.VMEM((CAPB,), jnp.int32),
    pltpu.VMEM((1, FLUSH), jnp.int32),
    pltpu.VMEM((1, FLUSH), jnp.int32),
    pltpu.VMEM((FLUSH, D), jnp.float32),
    pltpu.VMEM((L,), jnp.int32),
    pltpu.VMEM_SHARED((ACC_ROWS, D), jnp.float32),
    pltpu.SemaphoreType.DMA,
]


def _seg_pass(table, srci, dsti, out, sstg, dstg, csrc, cdst, srcrow, dstrow,
              rows, cnt, acc, sem):
    c = lax.axis_index("c")
    s = lax.axis_index("s")
    iota = lax.iota(jnp.int32, L)

    for sweep in range(NSWEEP):
        rbase = (NC * sweep + c) * R

        def _zbody(r, _):
            for q in range(D // L):
                rows[r, pl.ds(q * L, L)] = jnp.zeros((L,), jnp.float32)
            return 0
        lax.fori_loop(0, FLUSH, _zbody, 0)
        z0 = s * ZROWS
        for k in range(ZROWS // FLUSH):
            pltpu.sync_copy(rows, acc.at[pl.ds(z0 + k * FLUSH, FLUSH)])
        zr = ZROWS % FLUSH
        if zr:
            pltpu.sync_copy(rows.at[pl.ds(0, zr)],
                            acc.at[pl.ds(z0 + (ZROWS // FLUSH) * FLUSH, zr)])
        plsc.subcore_barrier()

        for blk in range(NBLK):
            nb = min(B, EPT - blk * B)
            ebase = s * EPT + blk * B
            pltpu.sync_copy(srci.at[pl.ds(ebase, nb)], sstg.at[pl.ds(0, nb)])
            pltpu.sync_copy(dsti.at[pl.ds(ebase, nb)], dstg.at[pl.ds(0, nb)])

            def _cbody(k, cur):
                p = k * L + iota
                dv = dstg[pl.ds(k * L, L)]
                sv = sstg[pl.ds(k * L, L)]
                m = (dv >= rbase) & (dv < rbase + R) & (p < nb)
                cs = plsc.cumsum(m.astype(jnp.int32))
                pos = cur + cs - 1
                plsc.store_scatter(cdst, [pos], dv - rbase, mask=m)
                plsc.store_scatter(csrc, [pos], sv, mask=m)
                return cur + plsc.all_reduce_population_count(m)
            cur = lax.fori_loop(0, (nb + L - 1) // L, _cbody,
                                jnp.zeros((L,), jnp.int32))
            cnt[pl.ds(0, L)] = cur
            count = cnt[pl.ds(0, L)][0]

            def _fbody(j, _):
                for q in range(FLUSH // L):
                    b = j * FLUSH + q * L
                    valid = (b + iota) < count
                    sv = csrc[pl.ds(b, L)]
                    dv = cdst[pl.ds(b, L)]
                    srcrow[0, pl.ds(q * L, L)] = jnp.where(valid, sv, iota)
                    dstrow[0, pl.ds(q * L, L)] = jnp.where(valid, dv, R + iota)
                pltpu.async_copy(table.at[srcrow.at[0]], rows, sem).wait()
                pltpu.sync_copy(rows, acc.at[dstrow.at[0]], add=True)
                return 0
            nfl = (count + FLUSH - 1) // FLUSH
            lax.fori_loop(0, nfl, _fbody, 0)

        plsc.subcore_barrier()

        o0 = s * OROWS
        pltpu.sync_copy(acc.at[pl.ds(o0, OROWS)],
                        out.at[pl.ds(rbase + o0, OROWS)])
        plsc.subcore_barrier()


@functools.partial(
    pl.kernel,
    mesh=_MESH,
    compiler_params=_SC_PARAMS,
    out_type=(jax.ShapeDtypeStruct((NP, D), jnp.float32),
              jax.ShapeDtypeStruct((NP, D), jnp.float32)),
    scratch_types=_SCRATCH,
)
def _segment_rows2(tab1, src1, dst1, tab2, src2, dst2, out1, out2, *scratch):
    _seg_pass(tab1, src1, dst1, out1, *scratch)
    _seg_pass(tab2, src2, dst2, out2, *scratch)


@functools.partial(
    pl.kernel,
    mesh=_MESH,
    compiler_params=_SC_PARAMS,
    out_type=jax.ShapeDtypeStruct((NP, D), jnp.float32),
    scratch_types=_SCRATCH,
)
def _segment_rows1(tab, src, dst, out, *scratch):
    _seg_pass(tab, src, dst, out, *scratch)


def _dense_kernel(agg_ref, x_ref, wl_ref, b_ref, wr_ref, o_ref, *, do_norm):
    h = (jnp.dot(agg_ref[...], wl_ref[...],
                 preferred_element_type=jnp.float32,
                 precision=lax.Precision.HIGHEST)
         + b_ref[...][None, :]
         + jnp.dot(x_ref[...], wr_ref[...],
                   preferred_element_type=jnp.float32,
                   precision=lax.Precision.HIGHEST))
    if do_norm:
        n = jnp.sqrt(jnp.sum(h * h, axis=-1, keepdims=True))
        h = h / jnp.maximum(n, 1e-12)
        h = jnp.maximum(h, 0.0)
    o_ref[...] = h


def _dense(agg, x, wl, b, wr, do_norm):
    n = agg.shape[0]
    blk = 1000
    grid = n // blk
    return pl.pallas_call(
        functools.partial(_dense_kernel, do_norm=do_norm),
        grid=(grid,),
        in_specs=[
            pl.BlockSpec((blk, D), lambda i: (i, 0)),
            pl.BlockSpec((blk, D), lambda i: (i, 0)),
            pl.BlockSpec((D, D), lambda i: (0, 0)),
            pl.BlockSpec((D,), lambda i: (0,)),
            pl.BlockSpec((D, D), lambda i: (0, 0)),
        ],
        out_specs=pl.BlockSpec((blk, D), lambda i: (i, 0)),
        out_shape=jax.ShapeDtypeStruct((n, D), jnp.float32),
    )(agg, x, wl, b, wr)


def kernel(x_note, x_user, edge_index_un, edge_index_nu, edge_attr_un,
           edge_attr_nu, W_l1_un, b_l1_un, W_r1_un, W_l1_nu, b_l1_nu, W_r1_nu,
           W_l2_un, b_l2_un, W_r2_un, W_l2_nu, b_l2_nu, W_r2_nu):
    src_un = edge_index_un[0]
    dst_un = edge_index_un[1]
    src_nu = edge_index_nu[0]
    dst_nu = edge_index_nu[1]

    agg_n1, agg_u1 = _segment_rows2(x_user, src_un, dst_un,
                                    x_note, src_nu, dst_nu)
    h_note = _dense(agg_n1[:N_NOTE], x_note, W_l1_un, b_l1_un, W_r1_un, True)
    h_user = _dense(agg_u1[:N_USER], x_user, W_l1_nu, b_l1_nu, W_r1_nu, True)
    agg_n2 = _segment_rows1(h_user, src_un, dst_un)[:N_NOTE]
    return _dense(agg_n2, h_note, W_l2_un, b_l2_un, W_r2_un, False)
